# Initial kernel scaffold; baseline (speedup 1.0000x reference)
import jax
import jax.numpy as jnp
from jax.experimental import pallas as pl


def kernel(x, edge_index, batch, W_gcn, b_gcn, W_lin, b_lin, W_att, b_att, W1, b1, W2, W3, b3):
    raise NotImplementedError("write your pallas kernel here")


# speedup vs baseline: 1.2472x; 1.2472x over previous
import jax
import jax.numpy as jnp
from jax.experimental import pallas as pl

_NEG_SLOPE = 0.2
_TOPK = 128


def _bs(shape, imap):
    return pl.BlockSpec(shape, imap)


def _conv_pre(x, w, deg2):
    n, d = x.shape
    bn = 1000 if n % 1000 == 0 else n
    grid = n // bn

    def body(x_ref, w_ref, deg_ref, o_ref):
        dinv = jax.lax.rsqrt(jnp.maximum(deg_ref[...], 1e-12))
        o_ref[...] = dinv * jnp.dot(x_ref[...], w_ref[...],
                                    preferred_element_type=jnp.float32)

    return pl.pallas_call(
        body, grid=(grid,),
        in_specs=[_bs((bn, d), lambda i: (i, 0)),
                  _bs((d, d), lambda i: (0, 0)),
                  _bs((bn, 1), lambda i: (i, 0))],
        out_specs=_bs((bn, d), lambda i: (i, 0)),
        out_shape=jax.ShapeDtypeStruct((n, d), jnp.float32),
    )(x, w, deg2)


def _conv_post(s1, deg2, b2, wb):
    n, d = s1.shape
    bn = 1000 if n % 1000 == 0 else n
    grid = n // bn

    def body(s_ref, deg_ref, b_ref, wb_ref, x1_ref, p_ref):
        dinv = jax.lax.rsqrt(jnp.maximum(deg_ref[...], 1e-12))
        x1 = jnp.maximum(dinv * s_ref[...] + b_ref[...], 0.0)
        x1_ref[...] = x1
        p_ref[...] = jnp.dot(x1, wb_ref[...], preferred_element_type=jnp.float32)

    return pl.pallas_call(
        body, grid=(grid,),
        in_specs=[_bs((bn, d), lambda i: (i, 0)),
                  _bs((bn, 1), lambda i: (i, 0)),
                  _bs((1, d), lambda i: (0, 0)),
                  _bs((d, 1), lambda i: (0, 0))],
        out_specs=[_bs((bn, d), lambda i: (i, 0)),
                   _bs((bn, 1), lambda i: (i, 0))],
        out_shape=[jax.ShapeDtypeStruct((n, d), jnp.float32),
                   jax.ShapeDtypeStruct((n, 1), jnp.float32)],
    )(s1, deg2, b2, wb)


def _attn_q(x_q, w_lin, bl2, wa, ba2):
    n, d = x_q.shape
    bn = 1000 if n % 1000 == 0 else n
    grid = n // bn

    def body(xq_ref, wl_ref, bl_ref, wa_ref, ba_ref, q_ref):
        h = jnp.dot(xq_ref[...], wl_ref[...],
                    preferred_element_type=jnp.float32) + bl_ref[...]
        q_ref[...] = jnp.dot(h, wa_ref[...],
                             preferred_element_type=jnp.float32) + ba_ref[...]

    return pl.pallas_call(
        body, grid=(grid,),
        in_specs=[_bs((bn, d), lambda i: (i, 0)),
                  _bs((d, d), lambda i: (0, 0)),
                  _bs((1, d), lambda i: (0, 0)),
                  _bs((d, 1), lambda i: (0, 0)),
                  _bs((1, 1), lambda i: (0, 0))],
        out_specs=_bs((bn, 1), lambda i: (i, 0)),
        out_shape=jax.ShapeDtypeStruct((n, 1), jnp.float32),
    )(x_q, w_lin, bl2, wa, ba2)


def _leconv_mm(x_new, w123):
    n, d = x_new.shape
    bn = 1000 if n % 1000 == 0 else n
    grid = n // bn

    def body(x_ref, w_ref, o_ref):
        o_ref[...] = jnp.dot(x_ref[...], w_ref[...],
                             preferred_element_type=jnp.float32)

    return pl.pallas_call(
        body, grid=(grid,),
        in_specs=[_bs((bn, d), lambda i: (i, 0)),
                  _bs((d, 8), lambda i: (0, 0))],
        out_specs=_bs((bn, 8), lambda i: (i, 0)),
        out_shape=jax.ShapeDtypeStruct((n, 8), jnp.float32),
    )(x_new, w123)


def _fitness(sa2, deg2, bv2, c2):
    n = sa2.shape[0]
    bn = 1000 if n % 1000 == 0 else n
    grid = n // bn

    def body(sa_ref, deg_ref, bv_ref, c_ref, o_ref):
        t = sa_ref[...] - deg_ref[...] * bv_ref[...] + c_ref[...]
        o_ref[...] = 1.0 / (1.0 + jnp.exp(-t))

    return pl.pallas_call(
        body, grid=(grid,),
        in_specs=[_bs((bn, 1), lambda i: (i, 0)) for _ in range(4)],
        out_specs=_bs((bn, 1), lambda i: (i, 0)),
        out_shape=jax.ShapeDtypeStruct((n, 1), jnp.float32),
    )(sa2, deg2, bv2, c2)


def _topk(fit2d, k):
    r, cdim = fit2d.shape
    kc = k // r

    def body(f_ref, v_ref, i_ref):
        f0 = f_ref[...]
        rix = jax.lax.broadcasted_iota(jnp.int32, (r, cdim), 0)
        cix = jax.lax.broadcasted_iota(jnp.int32, (r, cdim), 1)
        flat = rix * cdim + cix
        krix = jax.lax.broadcasted_iota(jnp.int32, (r, kc), 0)
        kcix = jax.lax.broadcasted_iota(jnp.int32, (r, kc), 1)
        kflat = krix * kc + kcix
        big = jnp.int32(2 ** 30)

        def step(j, carry):
            f, vals, idxs = carry
            m = jnp.max(f)
            cand = jnp.where(f == m, flat, big)
            sel = jnp.min(cand)
            hit = kflat == j
            vals = jnp.where(hit, m, vals)
            idxs = jnp.where(hit, sel, idxs)
            f = jnp.where(flat == sel, -jnp.inf, f)
            return f, vals, idxs

        _, vals, idxs = jax.lax.fori_loop(
            0, k, step,
            (f0, jnp.zeros((r, kc), jnp.float32), jnp.zeros((r, kc), jnp.int32)))
        v_ref[...] = vals
        i_ref[...] = idxs

    return pl.pallas_call(
        body,
        out_shape=[jax.ShapeDtypeStruct((r, kc), jnp.float32),
                   jax.ShapeDtypeStruct((r, kc), jnp.int32)],
    )(fit2d)


def _coarsen_mm(s_mat, t_mat, k):
    n = s_mat.shape[0]
    bn = 1000 if n % 1000 == 0 else n
    grid = n // bn

    def body(s_ref, t_ref, o_ref):
        i = pl.program_id(0)

        @pl.when(i == 0)
        def _():
            o_ref[...] = jnp.zeros_like(o_ref)

        o_ref[...] += jax.lax.dot_general(
            s_ref[...], t_ref[...], (((0,), (0,)), ((), ())),
            preferred_element_type=jnp.float32)

        @pl.when(i == grid - 1)
        def _():
            rix = jax.lax.broadcasted_iota(jnp.int32, (k, k), 0)
            cix = jax.lax.broadcasted_iota(jnp.int32, (k, k), 1)
            o_ref[...] = jnp.where(rix == cix, 0.0, o_ref[...])

    return pl.pallas_call(
        body, grid=(grid,),
        in_specs=[_bs((bn, k), lambda i: (i, 0)),
                  _bs((bn, k), lambda i: (i, 0))],
        out_specs=_bs((k, k), lambda i: (0, 0)),
        out_shape=jax.ShapeDtypeStruct((k, k), jnp.float32),
    )(s_mat, t_mat)


def _scale_rows(xg, fitk2):
    k, d = xg.shape

    def body(x_ref, f_ref, o_ref):
        o_ref[...] = x_ref[...] * f_ref[...]

    return pl.pallas_call(
        body,
        out_shape=jax.ShapeDtypeStruct((k, d), jnp.float32),
    )(xg, fitk2)


def kernel(x, edge_index, batch, W_gcn, b_gcn, W_lin, b_lin, W_att, b_att,
           W1, b1, W2, W3, b3):
    n, d = x.shape
    loop = jnp.arange(n, dtype=edge_index.dtype)
    row = jnp.concatenate([edge_index[0], loop])
    col = jnp.concatenate([edge_index[1], loop])

    deg = jax.ops.segment_sum(jnp.ones(row.shape[0], jnp.float32), col, n)
    deg2 = deg[:, None]
    z = _conv_pre(x, W_gcn, deg2)
    s1 = jax.ops.segment_sum(z[row], col, n)
    x1, p = _conv_post(s1, deg2, b_gcn[None, :], W_att[d:, :])

    x_q = jax.ops.segment_max(x1[row], col, n)
    q = _attn_q(x_q, W_lin, b_lin[None, :], W_att[:d, :], b_att[None, None, 0])
    score = q[:, 0][col] + p[:, 0][row]
    score = jnp.where(score > 0, score, _NEG_SLOPE * score)

    smax = jax.ops.segment_max(score, col, n)
    sexp = jnp.exp(score - smax[col])
    ssum = jax.ops.segment_sum(sexp, col, n)
    score = sexp / (ssum[col] + 1e-16)
    x_new = jax.ops.segment_sum(x1[row] * score[:, None], col, n)

    w123 = jnp.concatenate(
        [W1, W2, W3, jnp.zeros((d, 5), jnp.float32)], axis=1)
    abc = _leconv_mm(x_new, w123)
    a = abc[:, 0] + b1[0]
    bv = abc[:, 1:2]
    c = abc[:, 2:3] + b3[0]
    sa = jax.ops.segment_sum(a[row], col, n)[:, None]
    fitness = _fitness(sa, deg2, bv, c)[:, 0]

    vals, idxs = _topk(fitness.reshape(8, n // 8), _TOPK)
    fit_k = vals.reshape(-1)
    perm = idxs.reshape(-1)
    new_batch = batch[perm]

    pos = jnp.full((n,), -1, jnp.int32).at[perm].set(
        jnp.arange(_TOPK, dtype=jnp.int32))
    col_pos = pos[col]
    valid = col_pos >= 0
    s_mat = jnp.zeros((n, _TOPK), jnp.float32).at[
        row, jnp.where(valid, col_pos, 0)].add(jnp.where(valid, score, 0.0))
    t_mat = jax.ops.segment_sum(s_mat[col], row, n)
    a_new = _coarsen_mm(s_mat, t_mat, _TOPK)

    new_x = _scale_rows(x_new[perm], fit_k[:, None])
    return (new_x, a_new, perm, fitness, score, x1, new_batch)


# speedup vs baseline: 1.2887x; 1.0333x over previous
import jax
import jax.numpy as jnp
from jax.experimental import pallas as pl

_NEG_SLOPE = 0.2
_TOPK = 128


def _bs(shape, imap):
    return pl.BlockSpec(shape, imap)


def _conv_pre(x, w, deg2):
    n, d = x.shape
    bn = 1000 if n % 1000 == 0 else n
    grid = n // bn

    def body(x_ref, w_ref, deg_ref, o_ref):
        dinv = jax.lax.rsqrt(jnp.maximum(deg_ref[...], 1e-12))
        o_ref[...] = dinv * jnp.dot(x_ref[...], w_ref[...],
                                    preferred_element_type=jnp.float32)

    return pl.pallas_call(
        body, grid=(grid,),
        in_specs=[_bs((bn, d), lambda i: (i, 0)),
                  _bs((d, d), lambda i: (0, 0)),
                  _bs((bn, 1), lambda i: (i, 0))],
        out_specs=_bs((bn, d), lambda i: (i, 0)),
        out_shape=jax.ShapeDtypeStruct((n, d), jnp.float32),
    )(x, w, deg2)


def _conv_post(s1, deg2, b2, wb):
    n, d = s1.shape
    bn = 1000 if n % 1000 == 0 else n
    grid = n // bn

    def body(s_ref, deg_ref, b_ref, wb_ref, x1_ref, p_ref):
        dinv = jax.lax.rsqrt(jnp.maximum(deg_ref[...], 1e-12))
        x1 = jnp.maximum(dinv * s_ref[...] + b_ref[...], 0.0)
        x1_ref[...] = x1
        p_ref[...] = jnp.dot(x1, wb_ref[...], preferred_element_type=jnp.float32)

    return pl.pallas_call(
        body, grid=(grid,),
        in_specs=[_bs((bn, d), lambda i: (i, 0)),
                  _bs((bn, 1), lambda i: (i, 0)),
                  _bs((1, d), lambda i: (0, 0)),
                  _bs((d, 1), lambda i: (0, 0))],
        out_specs=[_bs((bn, d), lambda i: (i, 0)),
                   _bs((bn, 1), lambda i: (i, 0))],
        out_shape=[jax.ShapeDtypeStruct((n, d), jnp.float32),
                   jax.ShapeDtypeStruct((n, 1), jnp.float32)],
    )(s1, deg2, b2, wb)


def _attn_q(x_q, w_lin, bl2, wa, ba2):
    n, d = x_q.shape
    bn = 1000 if n % 1000 == 0 else n
    grid = n // bn

    def body(xq_ref, wl_ref, bl_ref, wa_ref, ba_ref, q_ref):
        h = jnp.dot(xq_ref[...], wl_ref[...],
                    preferred_element_type=jnp.float32) + bl_ref[...]
        q_ref[...] = jnp.dot(h, wa_ref[...],
                             preferred_element_type=jnp.float32) + ba_ref[...]

    return pl.pallas_call(
        body, grid=(grid,),
        in_specs=[_bs((bn, d), lambda i: (i, 0)),
                  _bs((d, d), lambda i: (0, 0)),
                  _bs((1, d), lambda i: (0, 0)),
                  _bs((d, 1), lambda i: (0, 0)),
                  _bs((1, 1), lambda i: (0, 0))],
        out_specs=_bs((bn, 1), lambda i: (i, 0)),
        out_shape=jax.ShapeDtypeStruct((n, 1), jnp.float32),
    )(x_q, w_lin, bl2, wa, ba2)


def _leconv_mm(x_new, w123):
    n, d = x_new.shape
    bn = 1000 if n % 1000 == 0 else n
    grid = n // bn

    def body(x_ref, w_ref, o_ref):
        o_ref[...] = jnp.dot(x_ref[...], w_ref[...],
                             preferred_element_type=jnp.float32)

    return pl.pallas_call(
        body, grid=(grid,),
        in_specs=[_bs((bn, d), lambda i: (i, 0)),
                  _bs((d, 8), lambda i: (0, 0))],
        out_specs=_bs((bn, 8), lambda i: (i, 0)),
        out_shape=jax.ShapeDtypeStruct((n, 8), jnp.float32),
    )(x_new, w123)


def _fitness(sa2, deg2, bv2, c2):
    n = sa2.shape[0]
    bn = 1000 if n % 1000 == 0 else n
    grid = n // bn

    def body(sa_ref, deg_ref, bv_ref, c_ref, o_ref):
        t = sa_ref[...] - deg_ref[...] * bv_ref[...] + c_ref[...]
        o_ref[...] = 1.0 / (1.0 + jnp.exp(-t))

    return pl.pallas_call(
        body, grid=(grid,),
        in_specs=[_bs((bn, 1), lambda i: (i, 0)) for _ in range(4)],
        out_specs=_bs((bn, 1), lambda i: (i, 0)),
        out_shape=jax.ShapeDtypeStruct((n, 1), jnp.float32),
    )(sa2, deg2, bv2, c2)


def _topk(fit2d, k):
    r, cdim = fit2d.shape
    kc = k // r

    def body(f_ref, v_ref, i_ref):
        f0 = f_ref[...]
        rix = jax.lax.broadcasted_iota(jnp.int32, (r, cdim), 0)
        cix = jax.lax.broadcasted_iota(jnp.int32, (r, cdim), 1)
        flat = rix * cdim + cix
        krix = jax.lax.broadcasted_iota(jnp.int32, (r, kc), 0)
        kcix = jax.lax.broadcasted_iota(jnp.int32, (r, kc), 1)
        kflat = krix * kc + kcix
        big = jnp.int32(2 ** 30)

        def step(j, carry):
            f, vals, idxs = carry
            m = jnp.max(f)
            cand = jnp.where(f == m, flat, big)
            sel = jnp.min(cand)
            hit = kflat == j
            vals = jnp.where(hit, m, vals)
            idxs = jnp.where(hit, sel, idxs)
            f = jnp.where(flat == sel, -jnp.inf, f)
            return f, vals, idxs

        _, vals, idxs = jax.lax.fori_loop(
            0, k, step,
            (f0, jnp.zeros((r, kc), jnp.float32), jnp.zeros((r, kc), jnp.int32)))
        v_ref[...] = vals
        i_ref[...] = idxs

    return pl.pallas_call(
        body,
        out_shape=[jax.ShapeDtypeStruct((r, kc), jnp.float32),
                   jax.ShapeDtypeStruct((r, kc), jnp.int32)],
    )(fit2d)


def _coarsen_mm(g_mat, h_mat, k):
    n = g_mat.shape[0]
    bn = 1000 if n % 1000 == 0 else n
    grid = n // bn

    def body(s_ref, t_ref, o_ref):
        i = pl.program_id(0)

        @pl.when(i == 0)
        def _():
            o_ref[...] = jnp.zeros_like(o_ref)

        o_ref[...] += jax.lax.dot_general(
            s_ref[...], t_ref[...], (((0,), (0,)), ((), ())),
            preferred_element_type=jnp.float32)

        @pl.when(i == grid - 1)
        def _():
            rix = jax.lax.broadcasted_iota(jnp.int32, (k, k), 0)
            cix = jax.lax.broadcasted_iota(jnp.int32, (k, k), 1)
            o_ref[...] = jnp.where(rix == cix, 0.0, o_ref[...])

    return pl.pallas_call(
        body, grid=(grid,),
        in_specs=[_bs((bn, k), lambda i: (i, 0)),
                  _bs((bn, k), lambda i: (i, 0))],
        out_specs=_bs((k, k), lambda i: (0, 0)),
        out_shape=jax.ShapeDtypeStruct((k, k), jnp.float32),
    )(g_mat, h_mat)


def _scale_rows(xg, fitk2):
    k, d = xg.shape

    def body(x_ref, f_ref, o_ref):
        o_ref[...] = x_ref[...] * f_ref[...]

    return pl.pallas_call(
        body,
        out_shape=jax.ShapeDtypeStruct((k, d), jnp.float32),
    )(xg, fitk2)


def kernel(x, edge_index, batch, W_gcn, b_gcn, W_lin, b_lin, W_att, b_att,
           W1, b1, W2, W3, b3):
    n, d = x.shape
    loop = jnp.arange(n, dtype=edge_index.dtype)
    row = jnp.concatenate([edge_index[0], loop])
    col = jnp.concatenate([edge_index[1], loop])

    deg = jax.ops.segment_sum(jnp.ones(row.shape[0], jnp.float32), col, n)
    deg2 = deg[:, None]
    z = _conv_pre(x, W_gcn, deg2)
    s1 = jax.ops.segment_sum(z[row], col, n)
    x1, p = _conv_post(s1, deg2, b_gcn[None, :], W_att[d:, :])

    x_q = jax.ops.segment_max(x1[row], col, n)
    q = _attn_q(x_q, W_lin, b_lin[None, :], W_att[:d, :], b_att[None, None, 0])
    score = q[:, 0][col] + p[:, 0][row]
    score = jnp.where(score > 0, score, _NEG_SLOPE * score)

    smax = jax.ops.segment_max(score, col, n)
    sexp = jnp.exp(score - smax[col])
    ssum = jax.ops.segment_sum(sexp, col, n)
    score = sexp / (ssum[col] + 1e-16)
    x_new = jax.ops.segment_sum(x1[row] * score[:, None], col, n)

    w123 = jnp.concatenate(
        [W1, W2, W3, jnp.zeros((d, 5), jnp.float32)], axis=1)
    abc = _leconv_mm(x_new, w123)
    a = abc[:, 0] + b1[0]
    bv = abc[:, 1:2]
    c = abc[:, 2:3] + b3[0]
    sa = jax.ops.segment_sum(a[row], col, n)[:, None]
    fitness = _fitness(sa, deg2, bv, c)[:, 0]

    vals, idxs = _topk(fitness.reshape(8, n // 8), _TOPK)
    fit_k = vals.reshape(-1)
    perm = idxs.reshape(-1)
    new_batch = batch[perm]

    pos = jnp.full((n,), -1, jnp.int32).at[perm].set(
        jnp.arange(_TOPK, dtype=jnp.int32))
    col_pos = pos[col]
    valid = col_pos >= 0
    s_mat = jnp.zeros((n, _TOPK), jnp.float32).at[
        row, jnp.where(valid, col_pos, 0)].add(jnp.where(valid, score, 0.0))
    a_new = _coarsen_mm(s_mat[row], s_mat[col], _TOPK)

    new_x = _scale_rows(x_new[perm], fit_k[:, None])
    return (new_x, a_new, perm, fitness, score, x1, new_batch)
